# baseline (speedup 1.0000x reference)
import functools

import jax
import jax.numpy as jnp
from jax import lax
from jax.experimental import pallas as pl
from jax.experimental.pallas import tpu as pltpu
from jax.experimental.pallas import tpu_sc as plsc

_N = 65536
_FLAT = _N * 3

_NC = 2
_NS = 16
_NW = _NC * _NS
_L = 16
_CHUNK = _FLAT // _NW

_mesh = plsc.VectorSubcoreMesh(core_axis_name="c", subcore_axis_name="s")


@functools.partial(
    pl.kernel,
    mesh=_mesh,
    out_type=jax.ShapeDtypeStruct((_FLAT,), jnp.float32),
    scratch_types=[
        pltpu.VMEM((_CHUNK,), jnp.float32),
        pltpu.VMEM((_CHUNK,), jnp.int32),
    ],
)
def _add_sc(x_hbm, i_hbm, o_hbm, xv, iv):
    wid = lax.axis_index("s") * _NC + lax.axis_index("c")
    base = wid * _CHUNK
    pltpu.sync_copy(x_hbm.at[pl.ds(base, _CHUNK)], xv)
    pltpu.sync_copy(i_hbm.at[pl.ds(base, _CHUNK)], iv)

    def step(j, carry):
        s = pl.ds(j * _L, _L)
        xv[s] = xv[s] + iv[s].astype(jnp.float32)
        return carry

    lax.fori_loop(0, _CHUNK // _L, step, 0)
    pltpu.sync_copy(xv, o_hbm.at[pl.ds(base, _CHUNK)])


def kernel(input_xyzs, query_xyz_index):
    out = _add_sc(input_xyzs.reshape(_FLAT), query_xyz_index.reshape(_FLAT))
    return out.reshape(_N, 3)
